# Initial kernel scaffold; baseline (speedup 1.0000x reference)
import jax
import jax.numpy as jnp
from jax.experimental import pallas as pl


def kernel(key, value, query, wq, wk, wv, wo):
    raise NotImplementedError("write your pallas kernel here")


# speedup vs baseline: 1.7651x; 1.7651x over previous
import functools

import jax
import jax.numpy as jnp
from jax.experimental import pallas as pl
from jax.experimental.pallas import tpu as pltpu

_MIB = 1024 * 1024


def _fused_kernel(q_ref, k_ref, v_ref, wqh_ref, wkh_ref, wf_ref,
                  out_ref, attn_ref, *, inv_heads):
    v = v_ref[...].astype(jnp.bfloat16)
    out_ref[...] = jnp.dot(v, wf_ref[...], preferred_element_type=jnp.float32)

    q = q_ref[...].astype(jnp.bfloat16)
    k = k_ref[...].astype(jnp.bfloat16)
    qh = jnp.dot(q, wqh_ref[...], preferred_element_type=jnp.float32)
    kh = jnp.dot(k, wkh_ref[...], preferred_element_type=jnp.float32)
    s = jax.lax.dot_general(qh.astype(jnp.bfloat16), kh.astype(jnp.bfloat16),
                            (((1,), (1,)), ((), ())),
                            preferred_element_type=jnp.float32)
    s = s - jnp.max(s, axis=-1, keepdims=True)
    e = jnp.exp(s)
    attn_ref[...] = e * (inv_heads / jnp.sum(e, axis=-1, keepdims=True))


def kernel(key, value, query, wq, wk, wv, wo):
    num_heads = 8
    B, Lk, Dk = key.shape
    _, Lv, Dv = value.shape
    _, Lq, _ = query.shape
    Dout = wo.shape[0]
    head_dim = Dk // num_heads
    lo = (num_heads - 1) * head_dim
    scale = head_dim ** (-0.5)

    hp = max(128, head_dim)
    wqh = jnp.zeros((Dk, hp), jnp.bfloat16).at[:, :head_dim].set(
        (scale * wq[lo:lo + head_dim, :]).T.astype(jnp.bfloat16))
    wkh = jnp.zeros((Dk, hp), jnp.bfloat16).at[:, :head_dim].set(
        wk[lo:lo + head_dim, :].T.astype(jnp.bfloat16))
    w_fused = (wv.T @ wo.T).astype(jnp.bfloat16)

    kern = functools.partial(_fused_kernel, inv_heads=1.0 / num_heads)

    in_bytes = 4 * (Lq * Dk + Lk * Dk + Lv * Dv)
    out_bytes = 4 * (Lv * Dout + Lq * Lk)
    w_bytes = 2 * (2 * Dk * hp + Dv * Dout)
    vmem = 2 * (in_bytes + out_bytes) + w_bytes + 6 * Lq * Lk * 4

    cost = pl.CostEstimate(
        flops=2 * B * (Lv * Dv * Dout + (Lq + Lk) * Dk * hp + Lq * Lk * hp),
        transcendentals=B * Lq * Lk,
        bytes_accessed=B * (in_bytes + out_bytes) + w_bytes)

    out, attn = pl.pallas_call(
        kern,
        out_shape=(jax.ShapeDtypeStruct((B, Lv, Dout), jnp.float32),
                   jax.ShapeDtypeStruct((B, Lq, Lk), jnp.float32)),
        grid=(B,),
        in_specs=[
            pl.BlockSpec((None, Lq, Dk), lambda b: (b, 0, 0)),
            pl.BlockSpec((None, Lk, Dk), lambda b: (b, 0, 0)),
            pl.BlockSpec((None, Lv, Dv), lambda b: (b, 0, 0)),
            pl.BlockSpec((Dk, hp), lambda b: (0, 0)),
            pl.BlockSpec((Dk, hp), lambda b: (0, 0)),
            pl.BlockSpec((Dv, Dout), lambda b: (0, 0)),
        ],
        out_specs=(pl.BlockSpec((None, Lv, Dout), lambda b: (b, 0, 0)),
                   pl.BlockSpec((None, Lq, Lk), lambda b: (b, 0, 0))),
        compiler_params=pltpu.CompilerParams(
            dimension_semantics=("parallel",),
            vmem_limit_bytes=int(min(max(vmem, 32 * _MIB), 64 * _MIB))),
        cost_estimate=cost,
    )(query, key, value, wqh, wkh, w_fused)
    return out, attn
